# baseline (speedup 1.0000x reference)
import jax
import jax.numpy as jnp
from jax import lax
from jax.experimental import pallas as pl
from jax.experimental.pallas import tpu as pltpu
from jax.experimental.pallas import tpu_sc as plsc

B, NF, L, D = 4096, 26, 50, 32
NW = 32
B_PER_W = B // NW
N_SUB = NF // 2
GROUP = 13


def _sc_body(idx_hbm, f_hbm, table_hbm, out_hbm,
             f_v, idx_v, rows_v, out_v, gsem0, gsem1, osem0, osem1):
    wid = lax.axis_index("s") * 2 + lax.axis_index("c")
    pltpu.sync_copy(f_hbm.at[pl.ds(0, L)], f_v)
    b0 = wid * B_PER_W
    gsems = (gsem0, gsem1)
    osems = (osem0, osem1)

    def gather_copies(slot):
        gsem = gsems[slot]
        return [pltpu.make_async_copy(
                    table_hbm.at[idx_v.at[slot].at[j]],
                    rows_v.at[slot].at[j], gsem)
                for j in range(NF)]

    def stage_fire(b, slot):
        pltpu.sync_copy(idx_hbm.at[b], idx_v.at[slot])
        for cp in gather_copies(slot):
            cp.start()

    def drain(slot):
        for cp in gather_copies(slot):
            cp.wait()

    def out_copy(b, slot):
        return pltpu.make_async_copy(out_v.at[slot], out_hbm.at[b], osems[slot])

    def compute(slot):
        def group_body(g, _):
            acc = [[jnp.zeros((16,), jnp.float32) for _ in range(2)]
                   for _ in range(GROUP)]
            for l in range(L):
                f0 = f_v[l, 0:16]
                f1 = f_v[l, 16:32]
                for k in range(GROUP):
                    nf = GROUP * g + k
                    acc[k][0] = acc[k][0] + rows_v[slot, nf, l, 0:16] * f0
                    acc[k][1] = acc[k][1] + rows_v[slot, nf, l, 16:32] * f1
            for k in range(GROUP):
                nf = GROUP * g + k
                out_v[slot, nf, 0:16] = acc[k][0]
                out_v[slot, nf, 16:32] = acc[k][1]
            return 0

        lax.fori_loop(0, NF // GROUP, group_body, 0)

    stage_fire(b0, 0)

    def pair_body(p, _):
        be = b0 + 2 * p
        bo = be + 1
        stage_fire(bo, 1)

        drain(0)

        @pl.when(p > 0)
        def _():
            out_copy(be, 0).wait()
        compute(0)
        out_copy(be, 0).start()

        @pl.when(p < B_PER_W // 2 - 1)
        def _():
            stage_fire(be + 2, 0)

        drain(1)

        @pl.when(p > 0)
        def _():
            out_copy(bo, 1).wait()
        compute(1)
        out_copy(bo, 1).start()
        return 0

    lax.fori_loop(0, B_PER_W // 2, pair_body, 0)
    out_copy(b0 + B_PER_W - 2, 0).wait()
    out_copy(b0 + B_PER_W - 1, 1).wait()


@jax.jit
def _encode(idx, f, table):
    mesh = plsc.VectorSubcoreMesh(core_axis_name="c", subcore_axis_name="s")
    run = pl.kernel(
        _sc_body,
        out_type=jax.ShapeDtypeStruct((B, NF, D), jnp.float32),
        mesh=mesh,
        scratch_types=[
            pltpu.VMEM((L, D), jnp.float32),
            pltpu.VMEM((2, NF, L), jnp.int32),
            pltpu.VMEM((2, NF, L, D), jnp.float32),
            pltpu.VMEM((2, NF, D), jnp.float32),
            pltpu.SemaphoreType.DMA,
            pltpu.SemaphoreType.DMA,
            pltpu.SemaphoreType.DMA,
            pltpu.SemaphoreType.DMA,
        ],
        compiler_params=pltpu.CompilerParams(use_tc_tiling_on_sc=False),
    )
    return run(idx, f, table)


def kernel(input_sequence, embedding_table, f):
    return _encode(input_sequence, f, embedding_table)


# speedup vs baseline: 1.7143x; 1.7143x over previous
import jax
import jax.numpy as jnp
from jax import lax
from jax.experimental import pallas as pl
from jax.experimental.pallas import tpu as pltpu
from jax.experimental.pallas import tpu_sc as plsc

B, NF, L, D = 4096, 26, 50, 32
NW = 32
BB = B // NW
NROWS = NF * L * NW
L_CHUNKS = (13, 13, 13, 11)
L_OFF = (0, 13, 26, 39)
LC_MAX = 13
GRP = 4


LPAD = 56


def _regroup_body(x_ref, o_ref):
    y = x_ref[...].reshape(L, BB)
    o_ref[...] = jnp.concatenate([y, y[0:LPAD - L]], axis=0)


@jax.jit
def _idx_regroup(t_idx):
    return pl.pallas_call(
        _regroup_body,
        out_shape=jax.ShapeDtypeStruct((NF * NW * LPAD, BB), jnp.int32),
        grid=(NF, NW),
        in_specs=[pl.BlockSpec((1, L, BB), lambda i, j: (i, 0, j))],
        out_specs=pl.BlockSpec((LPAD, BB), lambda i, j: (i * NW + j, 0)),
    )(t_idx)


def _sc_body(idx_hbm, f_hbm, table_hbm, out_hbm,
             f_v, idx_v, rows_v, out_v, gsem0, gsem1, osem0, osem1):
    wid = lax.axis_index("s") * 2 + lax.axis_index("c")
    pltpu.sync_copy(f_hbm.at[pl.ds(0, L)], f_v)
    b0 = wid * BB
    gsems = (gsem0, gsem1)
    osems = (osem0, osem1)
    iota = lax.iota(jnp.int32, 16)

    def gather_copies(islot, k, rslot):
        cl = L_CHUNKS[k]
        return [pltpu.make_async_copy(
                    table_hbm.at[idx_v.at[islot].at[L_OFF[k] + li]],
                    rows_v.at[rslot].at[li], gsems[rslot])
                for li in range(cl)]

    def stage_slab(nf, islot):
        pltpu.sync_copy(idx_hbm.at[pl.ds((nf * NW + wid) * LPAD, L)],
                        idx_v.at[islot])

    def fire(islot, k, rslot):
        for cp in gather_copies(islot, k, rslot):
            cp.start()

    def drain(islot, k, rslot):
        for cp in gather_copies(islot, k, rslot):
            cp.wait()

    def out_copy(nf, oslot):
        return pltpu.make_async_copy(
            out_v.at[oslot], out_hbm.at[nf, :, pl.ds(b0, BB)], osems[oslot])

    def compute(k, rslot, oslot):
        cl = L_CHUNKS[k]
        fregs = [(f_v[L_OFF[k] + li, 0:16], f_v[L_OFF[k] + li, 16:32])
                 for li in range(cl)]

        def group_body(g, _):
            acc = [[jnp.zeros((16,), jnp.float32) for _ in range(2)]
                   for _ in range(GRP)]
            for li in range(cl):
                f0, f1 = fregs[li]
                for s in range(GRP):
                    col = GRP * g + s
                    acc[s][0] = acc[s][0] + rows_v[rslot, li, col, 0:16] * f0
                    acc[s][1] = acc[s][1] + rows_v[rslot, li, col, 16:32] * f1
            for s in range(GRP):
                col = jnp.zeros((16,), jnp.int32) + (GRP * g + s)
                if k == 0:
                    plsc.store_scatter(out_v.at[oslot], [iota, col], acc[s][0])
                    plsc.store_scatter(out_v.at[oslot], [iota + 16, col], acc[s][1])
                else:
                    plsc.addupdate_scatter(out_v.at[oslot], [iota, col], acc[s][0])
                    plsc.addupdate_scatter(out_v.at[oslot], [iota + 16, col], acc[s][1])
            return 0

        lax.fori_loop(0, BB // GRP, group_body, 0)

    NPAIR = NF // 2

    stage_slab(0, 0)
    fire(0, 0, 0)

    def pair_body(p, _):
        nf0 = 2 * p
        nf1 = nf0 + 1
        stage_slab(nf1, 1)

        fire(0, 1, 1)
        drain(0, 0, 0)

        @pl.when(p > 0)
        def _():
            out_copy(nf0, 0).wait()
        compute(0, 0, 0)

        fire(0, 2, 0)
        drain(0, 1, 1)
        compute(1, 1, 0)

        fire(0, 3, 1)
        drain(0, 2, 0)
        compute(2, 0, 0)

        fire(1, 0, 0)
        drain(0, 3, 1)
        compute(3, 1, 0)
        out_copy(nf0, 0).start()

        @pl.when(p < NPAIR - 1)
        def _():
            stage_slab(nf0 + 2, 0)

        fire(1, 1, 1)
        drain(1, 0, 0)

        @pl.when(p > 0)
        def _():
            out_copy(nf1, 1).wait()
        compute(0, 0, 1)

        fire(1, 2, 0)
        drain(1, 1, 1)
        compute(1, 1, 1)

        fire(1, 3, 1)
        drain(1, 2, 0)
        compute(2, 0, 1)

        @pl.when(p < NPAIR - 1)
        def _():
            fire(0, 0, 0)
        drain(1, 3, 1)
        compute(3, 1, 1)
        out_copy(nf1, 1).start()
        return 0

    lax.fori_loop(0, NPAIR, pair_body, 0)
    out_copy(NF - 2, 0).wait()
    out_copy(NF - 1, 1).wait()


@jax.jit
def _encode(idx2, f, table):
    mesh = plsc.VectorSubcoreMesh(core_axis_name="c", subcore_axis_name="s")
    run = pl.kernel(
        _sc_body,
        out_type=jax.ShapeDtypeStruct((NF, D, B), jnp.float32),
        mesh=mesh,
        scratch_types=[
            pltpu.VMEM((L, D), jnp.float32),
            pltpu.VMEM((2, L, BB), jnp.int32),
            pltpu.VMEM((2, LC_MAX, BB, D), jnp.float32),
            pltpu.VMEM((2, D, BB), jnp.float32),
            pltpu.SemaphoreType.DMA,
            pltpu.SemaphoreType.DMA,
            pltpu.SemaphoreType.DMA,
            pltpu.SemaphoreType.DMA,
        ],
        compiler_params=pltpu.CompilerParams(use_tc_tiling_on_sc=False,
                                             needs_layout_passes=False),
    )
    return run(idx2, f, table)


def kernel(input_sequence, embedding_table, f):
    t_idx = jnp.transpose(input_sequence, (1, 2, 0))
    idx2 = _idx_regroup(t_idx)
    o = _encode(idx2, f, embedding_table)
    return jnp.transpose(o, (2, 0, 1))


# speedup vs baseline: 2.1761x; 1.2694x over previous
import jax
import jax.numpy as jnp
from jax import lax
from jax.experimental import pallas as pl
from jax.experimental.pallas import tpu as pltpu
from jax.experimental.pallas import tpu_sc as plsc

B, NF, L, D = 4096, 26, 50, 32
NW = 32
BB = B // NW
NROWS = NF * L * NW
L_CHUNKS = (13, 13, 13, 11)
L_OFF = (0, 13, 26, 39)
LC_MAX = 13
GRP = 4


LPAD = 56


def _regroup_body(x_ref, o_ref):
    x = x_ref[...]
    o_ref[...] = jnp.concatenate([x, x[:, 0:LPAD - L, :]], axis=1)


@jax.jit
def _idx_regroup(t_idx):
    out3 = pl.pallas_call(
        _regroup_body,
        out_shape=jax.ShapeDtypeStruct((NF, NW * LPAD, BB), jnp.int32),
        grid=(NW,),
        in_specs=[pl.BlockSpec((NF, L, BB), lambda j: (0, 0, j))],
        out_specs=pl.BlockSpec((NF, LPAD, BB), lambda j: (0, j, 0)),
    )(t_idx)
    return out3.reshape(NF * NW * LPAD, BB)


def _sc_body(idx_hbm, f_hbm, table_hbm, out_hbm,
             f_v, idx_v, rows_v, out_v, gsem0, gsem1, osem0, osem1):
    wid = lax.axis_index("s") * 2 + lax.axis_index("c")
    pltpu.sync_copy(f_hbm.at[pl.ds(0, L)], f_v)
    b0 = wid * BB
    gsems = (gsem0, gsem1)
    osems = (osem0, osem1)
    iota = lax.iota(jnp.int32, 16)

    def gather_copies(islot, k, rslot):
        cl = L_CHUNKS[k]
        return [pltpu.make_async_copy(
                    table_hbm.at[idx_v.at[islot].at[L_OFF[k] + li]],
                    rows_v.at[rslot].at[li], gsems[rslot])
                for li in range(cl)]

    def stage_slab(nf, islot):
        pltpu.sync_copy(idx_hbm.at[pl.ds((nf * NW + wid) * LPAD, L)],
                        idx_v.at[islot])

    def fire(islot, k, rslot):
        for cp in gather_copies(islot, k, rslot):
            cp.start()

    def drain(islot, k, rslot):
        for cp in gather_copies(islot, k, rslot):
            cp.wait()

    def out_copy(nf, oslot):
        return pltpu.make_async_copy(
            out_v.at[oslot], out_hbm.at[nf, :, pl.ds(b0, BB)], osems[oslot])

    def compute(k, rslot, oslot):
        cl = L_CHUNKS[k]
        fregs = [(f_v[L_OFF[k] + li, 0:16], f_v[L_OFF[k] + li, 16:32])
                 for li in range(cl)]

        def group_body(g, _):
            acc = [[jnp.zeros((16,), jnp.float32) for _ in range(2)]
                   for _ in range(GRP)]
            for li in range(cl):
                f0, f1 = fregs[li]
                for s in range(GRP):
                    col = GRP * g + s
                    acc[s][0] = acc[s][0] + rows_v[rslot, li, col, 0:16] * f0
                    acc[s][1] = acc[s][1] + rows_v[rslot, li, col, 16:32] * f1
            for s in range(GRP):
                col = jnp.zeros((16,), jnp.int32) + (GRP * g + s)
                if k == 0:
                    plsc.store_scatter(out_v.at[oslot], [iota, col], acc[s][0])
                    plsc.store_scatter(out_v.at[oslot], [iota + 16, col], acc[s][1])
                else:
                    plsc.addupdate_scatter(out_v.at[oslot], [iota, col], acc[s][0])
                    plsc.addupdate_scatter(out_v.at[oslot], [iota + 16, col], acc[s][1])
            return 0

        lax.fori_loop(0, BB // GRP, group_body, 0)

    NPAIR = NF // 2

    stage_slab(0, 0)
    fire(0, 0, 0)

    def pair_body(p, _):
        nf0 = 2 * p
        nf1 = nf0 + 1
        stage_slab(nf1, 1)

        fire(0, 1, 1)
        drain(0, 0, 0)

        @pl.when(p > 0)
        def _():
            out_copy(nf0, 0).wait()
        compute(0, 0, 0)

        fire(0, 2, 0)
        drain(0, 1, 1)
        compute(1, 1, 0)

        fire(0, 3, 1)
        drain(0, 2, 0)
        compute(2, 0, 0)

        fire(1, 0, 0)
        drain(0, 3, 1)
        compute(3, 1, 0)
        out_copy(nf0, 0).start()

        @pl.when(p < NPAIR - 1)
        def _():
            stage_slab(nf0 + 2, 0)

        fire(1, 1, 1)
        drain(1, 0, 0)

        @pl.when(p > 0)
        def _():
            out_copy(nf1, 1).wait()
        compute(0, 0, 1)

        fire(1, 2, 0)
        drain(1, 1, 1)
        compute(1, 1, 1)

        fire(1, 3, 1)
        drain(1, 2, 0)
        compute(2, 0, 1)

        @pl.when(p < NPAIR - 1)
        def _():
            fire(0, 0, 0)
        drain(1, 3, 1)
        compute(3, 1, 1)
        out_copy(nf1, 1).start()
        return 0

    lax.fori_loop(0, NPAIR, pair_body, 0)
    out_copy(NF - 2, 0).wait()
    out_copy(NF - 1, 1).wait()


@jax.jit
def _encode(idx2, f, table):
    mesh = plsc.VectorSubcoreMesh(core_axis_name="c", subcore_axis_name="s")
    run = pl.kernel(
        _sc_body,
        out_type=jax.ShapeDtypeStruct((NF, D, B), jnp.float32),
        mesh=mesh,
        scratch_types=[
            pltpu.VMEM((L, D), jnp.float32),
            pltpu.VMEM((2, L, BB), jnp.int32),
            pltpu.VMEM((2, LC_MAX, BB, D), jnp.float32),
            pltpu.VMEM((2, D, BB), jnp.float32),
            pltpu.SemaphoreType.DMA,
            pltpu.SemaphoreType.DMA,
            pltpu.SemaphoreType.DMA,
            pltpu.SemaphoreType.DMA,
        ],
        compiler_params=pltpu.CompilerParams(use_tc_tiling_on_sc=False,
                                             needs_layout_passes=False),
    )
    return run(idx2, f, table)


def kernel(input_sequence, embedding_table, f):
    t_idx = jnp.transpose(input_sequence, (1, 2, 0))
    idx2 = _idx_regroup(t_idx)
    o = _encode(idx2, f, embedding_table)
    return jnp.transpose(o, (2, 0, 1))
